# baseline (speedup 1.0000x reference)
import functools

import jax
import jax.numpy as jnp
from jax import lax
from jax.experimental import pallas as pl
from jax.experimental.pallas import tpu as pltpu
from jax.experimental.pallas import tpu_sc as plsc

_NPOINT = 512
_B = 8
_N = 32768
_NC = 2
_NS = 16
_L = 16
_OB = 8

_NEG = -1e30


_RC = 32


@functools.lru_cache(maxsize=None)
def _build(npoint, n):
    chunks = n // _L

    def _fps_sc(pts_hbm, out_hbm, px, py, pz, dist, obuf):
        c = lax.axis_index("c")
        s = lax.axis_index("s")
        wid = c * _NS + s

        @pl.when(wid < _B)
        def _():
            b = wid
            row = b * 3
            pltpu.sync_copy(pts_hbm.at[pl.ds(row * n, n)], px)
            pltpu.sync_copy(pts_hbm.at[pl.ds((row + 1) * n, n)], py)
            pltpu.sync_copy(pts_hbm.at[pl.ds((row + 2) * n, n)], pz)

            ii = lax.iota(jnp.int32, _L)

            def init_chunk(k, carry):
                dist[pl.ds(k * _L, _L)] = jnp.full((_L,), 1e10, jnp.float32)
                return carry

            lax.fori_loop(0, chunks - _RC, init_chunk, 0)

            cx0 = px[pl.ds(0, _L)][0]
            cy0 = py[pl.ds(0, _L)][0]
            cz0 = pz[pl.ds(0, _L)][0]

            def it_body(i, carry):
                far_g, cx, cy, cz, dregs = carry

                plsc.store_scatter(
                    obuf,
                    [jnp.full((_L,), lax.rem(i, _OB), jnp.int32)],
                    jnp.full((_L,), far_g, jnp.int32),
                    mask=ii == 0,
                )

                cxv = jnp.full((_L,), cx)
                cyv = jnp.full((_L,), cy)
                czv = jnp.full((_L,), cz)

                m = None
                midx = None
                ndregs = []
                for k in range(_RC):
                    sl = pl.ds(k * _L, _L)
                    dx = px[sl] - cxv
                    dy = py[sl] - cyv
                    dz = pz[sl] - czv
                    d = dx * dx + dy * dy + dz * dz
                    dn = jnp.minimum(dregs[k], d)
                    ndregs.append(dn)
                    if k == 0:
                        m = dn
                        midx = ii
                    else:
                        upd = dn > m
                        m = jnp.where(upd, dn, m)
                        midx = jnp.where(upd, k * _L + ii, midx)
                ndregs = tuple(ndregs)

                def chunk(k, mcarry):
                    mm, mi = mcarry
                    sl = pl.ds(k * _L, _L)
                    ddx = px[sl] - cxv
                    ddy = py[sl] - cyv
                    ddz = pz[sl] - czv
                    dd = ddx * ddx + ddy * ddy + ddz * ddz
                    dsl = pl.ds((k - _RC) * _L, _L)
                    dn = jnp.minimum(dist[dsl], dd)
                    dist[dsl] = dn
                    upd = dn > mm
                    mm = jnp.where(upd, dn, mm)
                    mi = jnp.where(upd, k * _L + ii, mi)
                    return (mm, mi)

                m, midx = lax.fori_loop(_RC, chunks, chunk, (m, midx))

                maxv = m[0]
                wloc = midx[0]
                for j in range(1, _L):
                    vj = m[j]
                    ij = midx[j]
                    better = (vj > maxv) | ((vj == maxv) & (ij < wloc))
                    maxv = jnp.where(better, vj, maxv)
                    wloc = jnp.where(better, ij, wloc)

                ivec = jnp.full((_L,), wloc, jnp.int32)
                gx = plsc.load_gather(px, [ivec])[0]
                gy = plsc.load_gather(py, [ivec])[0]
                gz = plsc.load_gather(pz, [ivec])[0]

                @pl.when(lax.rem(i, _OB) == _OB - 1)
                def _():
                    off = pl.multiple_of(b * npoint + (i - (_OB - 1)), _OB)
                    pltpu.sync_copy(obuf, out_hbm.at[pl.ds(off, _OB)])

                return (wloc, gx, gy, gz, ndregs)

            carry0 = (jnp.int32(0), cx0, cy0, cz0,
                      tuple(jnp.full((_L,), 1e10, jnp.float32)
                            for _ in range(_RC)))
            lax.fori_loop(0, npoint, it_body, carry0)

    return functools.partial(
        pl.kernel,
        out_type=jax.ShapeDtypeStruct((_B * npoint,), jnp.int32),
        compiler_params=pltpu.CompilerParams(needs_layout_passes=False),
        mesh=plsc.VectorSubcoreMesh(
            core_axis_name="c", subcore_axis_name="s",
            num_cores=_NC, num_subcores=_NS,
        ),
        scratch_types=[
            pltpu.VMEM((n,), jnp.float32),
            pltpu.VMEM((n,), jnp.float32),
            pltpu.VMEM((n,), jnp.float32),
            pltpu.VMEM((n - _RC * _L,), jnp.float32),
            pltpu.VMEM((_OB,), jnp.int32),
        ],
    )(_fps_sc)


def kernel(points, features, npoint):
    del features, npoint
    B, N, _ = points.shape
    pts_t = jnp.transpose(points, (0, 2, 1)).reshape(-1)
    return _build(_NPOINT, N)(pts_t).reshape(B, _NPOINT)


# speedup vs baseline: 11.2049x; 11.2049x over previous
import jax
import jax.numpy as jnp
from jax import lax
from jax.experimental import pallas as pl
from jax.experimental.pallas import tpu as pltpu

_NPOINT = 512


def _fps_kernel(pts_ref, out_ref, dist_ref):
    _, B, N = pts_ref.shape
    px = pts_ref[0]
    py = pts_ref[1]
    pz = pts_ref[2]
    lane = lax.broadcasted_iota(jnp.int32, (B, N), 1)
    ocol = lax.broadcasted_iota(jnp.int32, (B, _NPOINT), 1)

    dist_ref[...] = jnp.full((B, N), 1e10, dtype=jnp.float32)
    out_ref[...] = jnp.zeros((B, _NPOINT), dtype=jnp.int32)

    def body(i, far):
        out_ref[...] = jnp.where(ocol == i, far, out_ref[...])
        sel = lane == far
        cx = jnp.sum(jnp.where(sel, px, 0.0), axis=1, keepdims=True)
        cy = jnp.sum(jnp.where(sel, py, 0.0), axis=1, keepdims=True)
        cz = jnp.sum(jnp.where(sel, pz, 0.0), axis=1, keepdims=True)
        d = (px - cx) ** 2 + (py - cy) ** 2 + (pz - cz) ** 2
        dist = jnp.minimum(dist_ref[...], d)
        dist_ref[...] = dist
        mx = jnp.max(dist, axis=1, keepdims=True)
        far_new = jnp.min(jnp.where(dist == mx, lane, N), axis=1, keepdims=True)
        return far_new

    far0 = jnp.zeros((B, 1), dtype=jnp.int32)
    lax.fori_loop(0, _NPOINT, body, far0)


def kernel(points, features, npoint):
    del features, npoint
    B, N, _ = points.shape
    pts_t = jnp.transpose(points, (2, 0, 1))
    out = pl.pallas_call(
        _fps_kernel,
        out_shape=jax.ShapeDtypeStruct((B, _NPOINT), jnp.int32),
        scratch_shapes=[pltpu.VMEM((B, N), jnp.float32)],
    )(pts_t)
    return out
